# baseline (speedup 1.0000x reference)
import functools
import math

import jax
import jax.numpy as jnp
from jax import lax
from jax.experimental import pallas as pl
from jax.experimental.pallas import tpu as pltpu
from jax.experimental.pallas import tpu_sc as plsc

_AVG_LOG = math.log(33.0)
_NC = 32
_BN = 1000


def _wid():
    return lax.axis_index("s") * 2 + lax.axis_index("c")


def _mesh():
    return plsc.VectorSubcoreMesh(core_axis_name="c", subcore_axis_name="s")


@functools.lru_cache(maxsize=None)
def _make_sc_sums(n, c, E, W, nsplit):
    espan = E // nsplit
    n_win = espan // W
    assert n_win % 2 == 0

    out_type = [
        jax.ShapeDtypeStruct((_NC, c, n), jnp.float32),
        jax.ShapeDtypeStruct((_NC, c, n), jnp.float32),
    ]
    scratch = ([pltpu.VMEM((n,), jnp.float32)] * (3 * c)
               + [pltpu.VMEM((W,), jnp.int32)] * 4
               + [pltpu.SemaphoreType.DMA] * 4)

    @functools.partial(
        pl.kernel, out_type=out_type, mesh=_mesh(),
        compiler_params=pltpu.CompilerParams(needs_layout_passes=False),
        scratch_types=scratch,
    )
    def sums(bt_h, src_h, dst_h, sum_h, sq_h, *sc):
        bch = sc[0:c]
        acs = sc[c:2 * c]
        acq = sc[2 * c:3 * c]
        s0, s1, d0, d1 = sc[3 * c:3 * c + 4]
        sem_s0, sem_s1, sem_d0, sem_d1 = sc[3 * c + 4:]
        w = _wid()
        ebase = (w % nsplit) * espan
        for f in range(c):
            pltpu.sync_copy(bt_h.at[w // nsplit, f], bch[f])
        zero16 = jnp.zeros((16,), jnp.float32)

        def zinit(i, carry):
            for f in range(c):
                acs[f][pl.ds(i * 16, 16)] = zero16
                acq[f][pl.ds(i * 16, 16)] = zero16
            return carry

        lax.fori_loop(0, n // 16, zinit, 0)

        pltpu.async_copy(src_h.at[pl.ds(ebase, W)], s0, sem_s0)
        pltpu.async_copy(dst_h.at[pl.ds(ebase, W)], d0, sem_d0)
        pltpu.async_copy(src_h.at[pl.ds(ebase + W, W)], s1, sem_s1)
        pltpu.async_copy(dst_h.at[pl.ds(ebase + W, W)], d1, sem_d1)

        def process(sbuf, dbuf):
            @plsc.parallel_loop(0, W // 16, 1, unroll=4)
            def _(j):
                s16 = sbuf[pl.ds(j * 16, 16)]
                d16 = dbuf[pl.ds(j * 16, 16)]
                for f in range(c):
                    vals = plsc.load_gather(bch[f], [s16])
                    plsc.addupdate_scatter(acs[f], [d16], vals)
                    plsc.addupdate_scatter(acq[f], [d16], vals * vals)

        def pair(g, carry):
            for b, (sb, db, ss, sd) in enumerate(
                    ((s0, d0, sem_s0, sem_d0), (s1, d1, sem_s1, sem_d1))):
                wi = 2 * g + b
                off = ebase + wi * W
                pltpu.make_async_copy(src_h.at[pl.ds(off, W)], sb, ss).wait()
                pltpu.make_async_copy(dst_h.at[pl.ds(off, W)], db, sd).wait()
                process(sb, db)

                @pl.when(wi + 2 < n_win)
                def _():
                    pltpu.async_copy(src_h.at[pl.ds(off + 2 * W, W)], sb, ss)
                    pltpu.async_copy(dst_h.at[pl.ds(off + 2 * W, W)], db, sd)
            return carry

        lax.fori_loop(0, n_win // 2, pair, 0)
        for f in range(c):
            pltpu.sync_copy(acs[f], sum_h.at[w, f])
            pltpu.sync_copy(acq[f], sq_h.at[w, f])

    return sums


@functools.lru_cache(maxsize=None)
def _make_sc_max(n, c, E, W, with_deg, nsplit):
    espan = E // nsplit
    n_win = espan // W
    assert n_win % 2 == 0

    out_type = [jax.ShapeDtypeStruct((_NC, c, n), jnp.float32)]
    scratch = ([pltpu.VMEM((n,), jnp.float32)] * (2 * c)
               + [pltpu.VMEM((W,), jnp.int32)] * 4
               + [pltpu.SemaphoreType.DMA] * 4)
    if with_deg:
        out_type.append(jax.ShapeDtypeStruct((n,), jnp.float32))
        scratch = scratch + [pltpu.VMEM((n,), jnp.float32)]

    @functools.partial(
        pl.kernel, out_type=out_type, mesh=_mesh(), scratch_types=scratch,
        compiler_params=pltpu.CompilerParams(needs_layout_passes=False),
    )
    def mx(bt_h, src_h, dst_h, max_h, *rest):
        if with_deg:
            deg_h = rest[0]
            rest = rest[1:]
            dega = rest[-1]
            rest = rest[:-1]
        bch = rest[0:c]
        acm = rest[c:2 * c]
        s0, s1, d0, d1 = rest[2 * c:2 * c + 4]
        sem_s0, sem_s1, sem_d0, sem_d1 = rest[2 * c + 4:2 * c + 8]
        w = _wid()
        ebase = (w % nsplit) * espan
        for f in range(c):
            pltpu.sync_copy(bt_h.at[w // nsplit, f], bch[f])
        neg16 = jnp.full((16,), -1e30, jnp.float32)
        ones16 = jnp.ones((16,), jnp.float32)
        zero16 = jnp.zeros((16,), jnp.float32)

        def minit(i, carry):
            for f in range(c):
                acm[f][pl.ds(i * 16, 16)] = neg16
            return carry

        lax.fori_loop(0, n // 16, minit, 0)

        if with_deg:
            @pl.when(w == 0)
            def _():
                def dinit(i, carry):
                    dega[pl.ds(i * 16, 16)] = zero16
                    return carry
                lax.fori_loop(0, n // 16, dinit, 0)

        pltpu.async_copy(src_h.at[pl.ds(ebase, W)], s0, sem_s0)
        pltpu.async_copy(dst_h.at[pl.ds(ebase, W)], d0, sem_d0)
        pltpu.async_copy(src_h.at[pl.ds(ebase + W, W)], s1, sem_s1)
        pltpu.async_copy(dst_h.at[pl.ds(ebase + W, W)], d1, sem_d1)

        K = 5
        assert (W // 16) % K == 0

        def process(sbuf, dbuf):
            def bat(bi, carry2):
                base = bi * K
                d_l = [dbuf[pl.ds((base + t) * 16, 16)] for t in range(K)]
                s_l = [sbuf[pl.ds((base + t) * 16, 16)] for t in range(K)]
                v_l = [[plsc.load_gather(bch[f], [s_l[t]])
                        for f in range(c)] for t in range(K)]

                curs = [[plsc.load_gather(acm[f], [d_l[t]])
                         for f in range(c)] for t in range(K)]
                for t in range(K):
                    for f in range(c):
                        plsc.store_scatter(acm[f], [d_l[t]], v_l[t][f],
                                           mask=v_l[t][f] > curs[t][f])
                lost = None
                for t in range(K):
                    for f in range(c):
                        chk = plsc.load_gather(acm[f], [d_l[t]])
                        l = v_l[t][f] > chk
                        lost = l if lost is None else lost | l

                def fix(_go):
                    for t in range(K):
                        for f in range(c):
                            cur2 = plsc.load_gather(acm[f], [d_l[t]])
                            plsc.store_scatter(acm[f], [d_l[t]], v_l[t][f],
                                               mask=v_l[t][f] > cur2)
                    lost2 = None
                    for t in range(K):
                        for f in range(c):
                            chk2 = plsc.load_gather(acm[f], [d_l[t]])
                            l2 = v_l[t][f] > chk2
                            lost2 = l2 if lost2 is None else lost2 | l2
                    return jnp.any(lost2)

                lax.while_loop(lambda go: go, fix, jnp.any(lost))
                if with_deg:
                    @pl.when(w == 0)
                    def _():
                        for t in range(K):
                            plsc.addupdate_scatter(dega, [d_l[t]], ones16)
                return carry2

            lax.fori_loop(0, W // (16 * K), bat, 0)

        def pair(g, carry):
            for b, (sb, db, ss, sd) in enumerate(
                    ((s0, d0, sem_s0, sem_d0), (s1, d1, sem_s1, sem_d1))):
                wi = 2 * g + b
                off = ebase + wi * W
                pltpu.make_async_copy(src_h.at[pl.ds(off, W)], sb, ss).wait()
                pltpu.make_async_copy(dst_h.at[pl.ds(off, W)], db, sd).wait()
                process(sb, db)

                @pl.when(wi + 2 < n_win)
                def _():
                    pltpu.async_copy(src_h.at[pl.ds(off + 2 * W, W)], sb, ss)
                    pltpu.async_copy(dst_h.at[pl.ds(off + 2 * W, W)], db, sd)
            return carry

        lax.fori_loop(0, n_win // 2, pair, 0)
        for f in range(c):
            pltpu.sync_copy(acm[f], max_h.at[w, f])
        if with_deg:
            @pl.when(w == 0)
            def _():
                pltpu.sync_copy(dega, deg_h)

    return mx


def _full(shape):
    return pl.BlockSpec(shape, lambda i: (0,) * len(shape))


def _rows(bn, fdim):
    return pl.BlockSpec((bn, fdim), lambda i: (i, 0))


def _tc_pre_body(x_ref, wa_ref, wb_ref, b_ref, a_o, b_o):
    xb = x_ref[...]
    a_o[...] = jnp.dot(xb, wa_ref[...], preferred_element_type=jnp.float32)
    b_o[...] = (jnp.dot(xb, wb_ref[...], preferred_element_type=jnp.float32)
                + b_ref[...])


def _combine(x, A, S, Q, M, deg, postW, postb, linW, linb, g, b):
    degc = jnp.maximum(deg, 1.0)
    ssum = deg * A + S
    mean = ssum / degc
    mx = jnp.where(deg > 0, A + M, 0.0)
    s2 = deg * A * A + 2.0 * A * S + Q
    var = jnp.maximum(s2 / degc - mean * mean, 0.0)
    std = jnp.sqrt(var + 1e-5)
    agg = jnp.concatenate([mean, ssum, mx, std], axis=-1)
    amp = jnp.log(deg + 1.0) / _AVG_LOG
    att = _AVG_LOG / jnp.log(degc + 1.0)
    feat = jnp.concatenate([x, agg, agg * amp, agg * att], axis=-1)
    o = jnp.dot(feat, postW, preferred_element_type=jnp.float32) + postb
    o = jnp.dot(o, linW, preferred_element_type=jnp.float32) + linb
    mu = o.mean(axis=-1, keepdims=True)
    v = ((o - mu) ** 2).mean(axis=-1, keepdims=True)
    h = (o - mu) / jnp.sqrt(v + 1e-5) * g + b
    return jnp.maximum(h, 0.0)


def _tc_mid_body(x_ref, a_ref, s_ref, q_ref, m_ref, deg_ref, postw_ref,
                 postb_ref, linw_ref, linb_ref, g_ref, b_ref, wa1_ref,
                 wb1_ref, b1_ref, h_o, a1_o, b1_o):
    h = _combine(x_ref[...], a_ref[...], s_ref[...], q_ref[...], m_ref[...],
                 deg_ref[...], postw_ref[...], postb_ref[...], linw_ref[...],
                 linb_ref[...], g_ref[...], b_ref[...])
    h_o[...] = h
    a1_o[...] = jnp.dot(h, wa1_ref[...], preferred_element_type=jnp.float32)
    b1_o[...] = (jnp.dot(h, wb1_ref[...], preferred_element_type=jnp.float32)
                 + b1_ref[...])


def _tc_out_body(x_ref, a_ref, s_ref, q_ref, m_ref, deg_ref, d_ref,
                 postw_ref, postb_ref, linw_ref, linb_ref, g_ref, b_ref,
                 projw_ref, projb_ref, gp_ref, bp_ref, hw1_ref, hb1_ref,
                 hw2_ref, hb2_ref, ow_ref, ob_ref, y_o):
    h = _combine(x_ref[...], a_ref[...], s_ref[...], q_ref[...], m_ref[...],
                 deg_ref[...], postw_ref[...], postb_ref[...], linw_ref[...],
                 linb_ref[...], g_ref[...], b_ref[...])
    p = (jnp.dot(h, projw_ref[...], preferred_element_type=jnp.float32)
         + projb_ref[...])
    mu = p.mean(axis=-1, keepdims=True)
    v = ((p - mu) ** 2).mean(axis=-1, keepdims=True)
    phi = (p - mu) / jnp.sqrt(v + 1e-5) * gp_ref[...] + bp_ref[...]
    hw1 = hw1_ref[...]
    hb1 = hb1_ref[...]
    hw2 = hw2_ref[...]
    hb2 = hb2_ref[...]
    ow = ow_ref[...]
    ob = ob_ref[...]
    d = d_ref[...]
    acc = jnp.zeros((phi.shape[0], 1), jnp.float32)
    for k in range(hw1.shape[0]):
        z = jnp.maximum(
            jnp.dot(phi, hw1[k], preferred_element_type=jnp.float32)
            + hb1[k][None, :], 0.0)
        z = jnp.maximum(
            jnp.dot(z, hw2[k], preferred_element_type=jnp.float32)
            + hb2[k][None, :], 0.0)
        yk = jnp.dot(z, ow[k], preferred_element_type=jnp.float32) + ob[k][None, :]
        acc = acc + jnp.where(d == k, yk, 0.0)
    y_o[...] = acc


def _edge_stats(B, src, dst, n, F, E, with_deg, nsplit):
    nchunk = _NC // nsplit
    c = F // nchunk
    Bt = B.reshape(n, nchunk, c).transpose(1, 2, 0)
    Sc, Qc = _make_sc_sums(n, c, E, 2000, nsplit)(Bt, src, dst)
    if with_deg:
        Mc, deg = _make_sc_max(n, c, E, 4000, True, nsplit)(Bt, src, dst)
    else:
        (Mc,) = _make_sc_max(n, c, E, 4000, False, nsplit)(Bt, src, dst)
        deg = None

    def unchunk(Xc, part):
        Xp = Xc.reshape(nchunk, nsplit, c, n)[:, part]
        return Xp.transpose(2, 0, 1).reshape(n, F)

    parts = tuple(tuple(unchunk(Xc, p) for p in range(nsplit))
                  for Xc in (Sc, Qc, Mc))
    return parts[0], parts[1], parts[2], deg


def kernel(x, edge_index, D, pre_W0, pre_b0, post_W0, post_b0, lin_W0,
           lin_b0, ln_g0, ln_b0, pre_W1, pre_b1, post_W1, post_b1, lin_W1,
           lin_b1, ln_g1, ln_b1, proj_W, proj_b, ln_gp, ln_bp, hW1, hb1,
           hW2, hb2, oW, ob):
    n, fin = x.shape
    E = edge_index.shape[1]
    src, dst = edge_index[0], edge_index[1]
    grid = (n // _BN,)
    row = lambda v: v.reshape(1, -1)

    f0 = pre_W0.shape[1]
    A0, B0 = pl.pallas_call(
        _tc_pre_body,
        grid=grid,
        in_specs=[_rows(_BN, fin), _full((fin, f0)), _full((fin, f0)),
                  _full((1, f0))],
        out_specs=[_rows(_BN, f0), _rows(_BN, f0)],
        out_shape=[jax.ShapeDtypeStruct((n, f0), jnp.float32),
                   jax.ShapeDtypeStruct((n, f0), jnp.float32)],
    )(x, pre_W0[:fin], pre_W0[fin:], row(pre_b0))

    (S0,), (Q0,), (M0,), deg = _edge_stats(B0, src, dst, n, f0, E, True, 1)
    deg2 = deg.reshape(n, 1)

    f1 = pre_W1.shape[1]
    pdim0 = post_W0.shape[0]
    h0, A1, B1 = pl.pallas_call(
        _tc_mid_body,
        grid=grid,
        in_specs=[_rows(_BN, fin), _rows(_BN, f0), _rows(_BN, f0),
                  _rows(_BN, f0), _rows(_BN, f0), _rows(_BN, 1),
                  _full((pdim0, f1)), _full((1, f1)), _full((f1, f1)),
                  _full((1, f1)), _full((1, f1)), _full((1, f1)),
                  _full((f1, f1)), _full((f1, f1)), _full((1, f1))],
        out_specs=[_rows(_BN, f1), _rows(_BN, f1), _rows(_BN, f1)],
        out_shape=[jax.ShapeDtypeStruct((n, f1), jnp.float32),
                   jax.ShapeDtypeStruct((n, f1), jnp.float32),
                   jax.ShapeDtypeStruct((n, f1), jnp.float32)],
    )(x, A0, S0, Q0, M0, deg2, post_W0, row(post_b0), lin_W0, row(lin_b0),
      row(ln_g0), row(ln_b0), pre_W1[:f1], pre_W1[f1:], row(pre_b1))

    (S1,), (Q1,), (M1,), _ = _edge_stats(B1, src, dst, n, f1, E, False, 1)

    f2 = lin_W1.shape[1]
    pdim1 = post_W1.shape[0]
    fp = proj_W.shape[1]
    K = hW1.shape[0]
    hh = hW1.shape[2]
    y = pl.pallas_call(
        _tc_out_body,
        grid=grid,
        in_specs=[_rows(_BN, f1), _rows(_BN, f1), _rows(_BN, f1),
                  _rows(_BN, f1), _rows(_BN, f1), _rows(_BN, 1),
                  _rows(_BN, 1),
                  _full((pdim1, f2)), _full((1, f2)), _full((f2, f2)),
                  _full((1, f2)), _full((1, f2)), _full((1, f2)),
                  _full((f2, fp)), _full((1, fp)), _full((1, fp)),
                  _full((1, fp)),
                  _full((K, fp, hh)), _full((K, hh)), _full((K, hh, fp)),
                  _full((K, fp)), _full((K, fp, 1)), _full((K, 1))],
        out_specs=[_rows(_BN, 1)],
        out_shape=[jax.ShapeDtypeStruct((n, 1), jnp.float32)],
    )(h0, A1, S1, Q1, M1, deg2, D.reshape(n, 1), post_W1, row(post_b1),
      lin_W1, row(lin_b1), row(ln_g1), row(ln_b1), proj_W, row(proj_b),
      row(ln_gp), row(ln_bp), hW1, hb1, hW2, hb2, oW, ob)[0]
    return y.reshape(n)


# speedup vs baseline: 1.0296x; 1.0296x over previous
import functools
import math

import jax
import jax.numpy as jnp
from jax import lax
from jax.experimental import pallas as pl
from jax.experimental.pallas import tpu as pltpu
from jax.experimental.pallas import tpu_sc as plsc

_AVG_LOG = math.log(33.0)
_NC = 32
_BN = 1000


def _wid():
    return lax.axis_index("s") * 2 + lax.axis_index("c")


def _mesh():
    return plsc.VectorSubcoreMesh(core_axis_name="c", subcore_axis_name="s")


@functools.lru_cache(maxsize=None)
def _make_sc_sums(n, c, E, W, nsplit):
    espan = E // nsplit
    n_win = espan // W
    assert n_win % 2 == 0 and c % 2 == 0
    cp = c // 2

    out_type = [
        jax.ShapeDtypeStruct((_NC, c, n), jnp.float32),
        jax.ShapeDtypeStruct((_NC, c, n), jnp.float32),
    ]
    scratch = ([pltpu.VMEM((n,), jnp.int32)] * cp
               + [pltpu.VMEM((n,), jnp.float32)] * (2 * c)
               + [pltpu.VMEM((W,), jnp.int32)] * 4
               + [pltpu.SemaphoreType.DMA] * 4)

    @functools.partial(
        pl.kernel, out_type=out_type, mesh=_mesh(),
        compiler_params=pltpu.CompilerParams(needs_layout_passes=False),
        scratch_types=scratch,
    )
    def sums(bt_h, src_h, dst_h, sum_h, sq_h, *sc):
        bch = sc[0:cp]
        acs = sc[cp:cp + c]
        acq = sc[cp + c:cp + 2 * c]
        s0, s1, d0, d1 = sc[cp + 2 * c:cp + 2 * c + 4]
        sem_s0, sem_s1, sem_d0, sem_d1 = sc[cp + 2 * c + 4:]
        w = _wid()
        ebase = (w % nsplit) * espan
        for fp in range(cp):
            pltpu.sync_copy(bt_h.at[w // nsplit, fp], bch[fp])
        zero16 = jnp.zeros((16,), jnp.float32)

        def zinit(i, carry):
            for f in range(c):
                acs[f][pl.ds(i * 16, 16)] = zero16
                acq[f][pl.ds(i * 16, 16)] = zero16
            return carry

        lax.fori_loop(0, n // 16, zinit, 0)

        pltpu.async_copy(src_h.at[pl.ds(ebase, W)], s0, sem_s0)
        pltpu.async_copy(dst_h.at[pl.ds(ebase, W)], d0, sem_d0)
        pltpu.async_copy(src_h.at[pl.ds(ebase + W, W)], s1, sem_s1)
        pltpu.async_copy(dst_h.at[pl.ds(ebase + W, W)], d1, sem_d1)

        def process(sbuf, dbuf):
            @plsc.parallel_loop(0, W // 16, 1, unroll=4)
            def _(j):
                s16 = sbuf[pl.ds(j * 16, 16)]
                d16 = dbuf[pl.ds(j * 16, 16)]
                for fp in range(cp):
                    wv = plsc.load_gather(bch[fp], [s16])
                    ab = plsc.bitcast(wv, jnp.bfloat16)
                    va, vb = plsc.unpack(
                        ab, format=plsc.PackFormat.INTERLEAVED,
                        preferred_element_type=jnp.float32)
                    plsc.addupdate_scatter(acs[2 * fp], [d16], va)
                    plsc.addupdate_scatter(acq[2 * fp], [d16], va * va)
                    plsc.addupdate_scatter(acs[2 * fp + 1], [d16], vb)
                    plsc.addupdate_scatter(acq[2 * fp + 1], [d16], vb * vb)

        def pair(g, carry):
            for b, (sb, db, ss, sd) in enumerate(
                    ((s0, d0, sem_s0, sem_d0), (s1, d1, sem_s1, sem_d1))):
                wi = 2 * g + b
                off = ebase + wi * W
                pltpu.make_async_copy(src_h.at[pl.ds(off, W)], sb, ss).wait()
                pltpu.make_async_copy(dst_h.at[pl.ds(off, W)], db, sd).wait()
                process(sb, db)

                @pl.when(wi + 2 < n_win)
                def _():
                    pltpu.async_copy(src_h.at[pl.ds(off + 2 * W, W)], sb, ss)
                    pltpu.async_copy(dst_h.at[pl.ds(off + 2 * W, W)], db, sd)
            return carry

        lax.fori_loop(0, n_win // 2, pair, 0)
        for f in range(c):
            pltpu.sync_copy(acs[f], sum_h.at[w, f])
            pltpu.sync_copy(acq[f], sq_h.at[w, f])

    return sums


@functools.lru_cache(maxsize=None)
def _make_sc_max(n, c, E, W, with_deg, nsplit):
    espan = E // nsplit
    n_win = espan // W
    assert n_win % 2 == 0

    out_type = [jax.ShapeDtypeStruct((_NC, c, n), jnp.float32)]
    scratch = ([pltpu.VMEM((n,), jnp.float32)] * (2 * c)
               + [pltpu.VMEM((W,), jnp.int32)] * 4
               + [pltpu.SemaphoreType.DMA] * 4)
    if with_deg:
        out_type.append(jax.ShapeDtypeStruct((n,), jnp.float32))
        scratch = scratch + [pltpu.VMEM((n,), jnp.float32)]

    @functools.partial(
        pl.kernel, out_type=out_type, mesh=_mesh(), scratch_types=scratch,
        compiler_params=pltpu.CompilerParams(needs_layout_passes=False),
    )
    def mx(bt_h, src_h, dst_h, max_h, *rest):
        if with_deg:
            deg_h = rest[0]
            rest = rest[1:]
            dega = rest[-1]
            rest = rest[:-1]
        bch = rest[0:c]
        acm = rest[c:2 * c]
        s0, s1, d0, d1 = rest[2 * c:2 * c + 4]
        sem_s0, sem_s1, sem_d0, sem_d1 = rest[2 * c + 4:2 * c + 8]
        w = _wid()
        ebase = (w % nsplit) * espan
        for f in range(c):
            pltpu.sync_copy(bt_h.at[w // nsplit, f], bch[f])
        neg16 = jnp.full((16,), -1e30, jnp.float32)
        ones16 = jnp.ones((16,), jnp.float32)
        zero16 = jnp.zeros((16,), jnp.float32)

        def minit(i, carry):
            for f in range(c):
                acm[f][pl.ds(i * 16, 16)] = neg16
            return carry

        lax.fori_loop(0, n // 16, minit, 0)

        if with_deg:
            @pl.when(w == 0)
            def _():
                def dinit(i, carry):
                    dega[pl.ds(i * 16, 16)] = zero16
                    return carry
                lax.fori_loop(0, n // 16, dinit, 0)

        pltpu.async_copy(src_h.at[pl.ds(ebase, W)], s0, sem_s0)
        pltpu.async_copy(dst_h.at[pl.ds(ebase, W)], d0, sem_d0)
        pltpu.async_copy(src_h.at[pl.ds(ebase + W, W)], s1, sem_s1)
        pltpu.async_copy(dst_h.at[pl.ds(ebase + W, W)], d1, sem_d1)

        K = max(2, 16 // c)
        assert (W // 16) % K == 0

        def process(sbuf, dbuf):
            def bat(bi, carry2):
                base = bi * K
                d_l = [dbuf[pl.ds((base + t) * 16, 16)] for t in range(K)]
                s_l = [sbuf[pl.ds((base + t) * 16, 16)] for t in range(K)]
                v_l = [[plsc.load_gather(bch[f], [s_l[t]])
                        for f in range(c)] for t in range(K)]

                curs = [[plsc.load_gather(acm[f], [d_l[t]])
                         for f in range(c)] for t in range(K)]
                for t in range(K):
                    for f in range(c):
                        plsc.store_scatter(acm[f], [d_l[t]], v_l[t][f],
                                           mask=v_l[t][f] > curs[t][f])
                lost = None
                for t in range(K):
                    for f in range(c):
                        chk = plsc.load_gather(acm[f], [d_l[t]])
                        l = v_l[t][f] > chk
                        lost = l if lost is None else lost | l

                def fix(_go):
                    for t in range(K):
                        for f in range(c):
                            cur2 = plsc.load_gather(acm[f], [d_l[t]])
                            plsc.store_scatter(acm[f], [d_l[t]], v_l[t][f],
                                               mask=v_l[t][f] > cur2)
                    lost2 = None
                    for t in range(K):
                        for f in range(c):
                            chk2 = plsc.load_gather(acm[f], [d_l[t]])
                            l2 = v_l[t][f] > chk2
                            lost2 = l2 if lost2 is None else lost2 | l2
                    return jnp.any(lost2)

                lax.while_loop(lambda go: go, fix, jnp.any(lost))
                if with_deg:
                    @pl.when(w == 0)
                    def _():
                        for t in range(K):
                            plsc.addupdate_scatter(dega, [d_l[t]], ones16)
                return carry2

            lax.fori_loop(0, W // (16 * K), bat, 0)

        def pair(g, carry):
            for b, (sb, db, ss, sd) in enumerate(
                    ((s0, d0, sem_s0, sem_d0), (s1, d1, sem_s1, sem_d1))):
                wi = 2 * g + b
                off = ebase + wi * W
                pltpu.make_async_copy(src_h.at[pl.ds(off, W)], sb, ss).wait()
                pltpu.make_async_copy(dst_h.at[pl.ds(off, W)], db, sd).wait()
                process(sb, db)

                @pl.when(wi + 2 < n_win)
                def _():
                    pltpu.async_copy(src_h.at[pl.ds(off + 2 * W, W)], sb, ss)
                    pltpu.async_copy(dst_h.at[pl.ds(off + 2 * W, W)], db, sd)
            return carry

        lax.fori_loop(0, n_win // 2, pair, 0)
        for f in range(c):
            pltpu.sync_copy(acm[f], max_h.at[w, f])
        if with_deg:
            @pl.when(w == 0)
            def _():
                pltpu.sync_copy(dega, deg_h)

    return mx


def _full(shape):
    return pl.BlockSpec(shape, lambda i: (0,) * len(shape))


def _rows(bn, fdim):
    return pl.BlockSpec((bn, fdim), lambda i: (i, 0))


def _tc_pre_body(x_ref, wa_ref, wb_ref, b_ref, a_o, b_o):
    xb = x_ref[...]
    a_o[...] = jnp.dot(xb, wa_ref[...], preferred_element_type=jnp.float32)
    b_o[...] = (jnp.dot(xb, wb_ref[...], preferred_element_type=jnp.float32)
                + b_ref[...])


def _combine(x, A, S, Q, M, deg, postW, postb, linW, linb, g, b):
    degc = jnp.maximum(deg, 1.0)
    ssum = deg * A + S
    mean = ssum / degc
    mx = jnp.where(deg > 0, A + M, 0.0)
    s2 = deg * A * A + 2.0 * A * S + Q
    var = jnp.maximum(s2 / degc - mean * mean, 0.0)
    std = jnp.sqrt(var + 1e-5)
    agg = jnp.concatenate([mean, ssum, mx, std], axis=-1)
    amp = jnp.log(deg + 1.0) / _AVG_LOG
    att = _AVG_LOG / jnp.log(degc + 1.0)
    feat = jnp.concatenate([x, agg, agg * amp, agg * att], axis=-1)
    o = jnp.dot(feat, postW, preferred_element_type=jnp.float32) + postb
    o = jnp.dot(o, linW, preferred_element_type=jnp.float32) + linb
    mu = o.mean(axis=-1, keepdims=True)
    v = ((o - mu) ** 2).mean(axis=-1, keepdims=True)
    h = (o - mu) / jnp.sqrt(v + 1e-5) * g + b
    return jnp.maximum(h, 0.0)


def _tc_mid_body(x_ref, a_ref, s_ref, q_ref, m_ref, deg_ref, postw_ref,
                 postb_ref, linw_ref, linb_ref, g_ref, b_ref, wa1_ref,
                 wb1_ref, b1_ref, h_o, a1_o, b1_o):
    h = _combine(x_ref[...], a_ref[...], s_ref[...], q_ref[...], m_ref[...],
                 deg_ref[...], postw_ref[...], postb_ref[...], linw_ref[...],
                 linb_ref[...], g_ref[...], b_ref[...])
    h_o[...] = h
    a1_o[...] = jnp.dot(h, wa1_ref[...], preferred_element_type=jnp.float32)
    b1_o[...] = (jnp.dot(h, wb1_ref[...], preferred_element_type=jnp.float32)
                 + b1_ref[...])


def _tc_out_body(x_ref, a_ref, s_ref, q_ref, m_ref, deg_ref, d_ref,
                 postw_ref, postb_ref, linw_ref, linb_ref, g_ref, b_ref,
                 projw_ref, projb_ref, gp_ref, bp_ref, hw1_ref, hb1_ref,
                 hw2_ref, hb2_ref, ow_ref, ob_ref, y_o):
    h = _combine(x_ref[...], a_ref[...], s_ref[...], q_ref[...], m_ref[...],
                 deg_ref[...], postw_ref[...], postb_ref[...], linw_ref[...],
                 linb_ref[...], g_ref[...], b_ref[...])
    p = (jnp.dot(h, projw_ref[...], preferred_element_type=jnp.float32)
         + projb_ref[...])
    mu = p.mean(axis=-1, keepdims=True)
    v = ((p - mu) ** 2).mean(axis=-1, keepdims=True)
    phi = (p - mu) / jnp.sqrt(v + 1e-5) * gp_ref[...] + bp_ref[...]
    hw1 = hw1_ref[...]
    hb1 = hb1_ref[...]
    hw2 = hw2_ref[...]
    hb2 = hb2_ref[...]
    ow = ow_ref[...]
    ob = ob_ref[...]
    d = d_ref[...]
    acc = jnp.zeros((phi.shape[0], 1), jnp.float32)
    for k in range(hw1.shape[0]):
        z = jnp.maximum(
            jnp.dot(phi, hw1[k], preferred_element_type=jnp.float32)
            + hb1[k][None, :], 0.0)
        z = jnp.maximum(
            jnp.dot(z, hw2[k], preferred_element_type=jnp.float32)
            + hb2[k][None, :], 0.0)
        yk = jnp.dot(z, ow[k], preferred_element_type=jnp.float32) + ob[k][None, :]
        acc = acc + jnp.where(d == k, yk, 0.0)
    y_o[...] = acc


def _edge_stats(B, src, dst, n, F, E, with_deg, nsplit):
    nchunk = _NC // nsplit
    c = F // nchunk
    Bt = B.reshape(n, nchunk, c).transpose(1, 2, 0)
    Bp = jax.lax.bitcast_convert_type(
        B.astype(jnp.bfloat16).reshape(n, F // 2, 2), jnp.int32)
    Btp = Bp.reshape(n, nchunk, c // 2).transpose(1, 2, 0)
    Sc, Qc = _make_sc_sums(n, c, E, 2000, nsplit)(Btp, src, dst)
    if with_deg:
        Mc, deg = _make_sc_max(n, c, E, 3200, True, nsplit)(Bt, src, dst)
    else:
        (Mc,) = _make_sc_max(n, c, E, 3200, False, nsplit)(Bt, src, dst)
        deg = None

    def unchunk(Xc, part):
        Xp = Xc.reshape(nchunk, nsplit, c, n)[:, part]
        return Xp.transpose(2, 0, 1).reshape(n, F)

    parts = tuple(tuple(unchunk(Xc, p) for p in range(nsplit))
                  for Xc in (Sc, Qc, Mc))
    return parts[0], parts[1], parts[2], deg


def kernel(x, edge_index, D, pre_W0, pre_b0, post_W0, post_b0, lin_W0,
           lin_b0, ln_g0, ln_b0, pre_W1, pre_b1, post_W1, post_b1, lin_W1,
           lin_b1, ln_g1, ln_b1, proj_W, proj_b, ln_gp, ln_bp, hW1, hb1,
           hW2, hb2, oW, ob):
    n, fin = x.shape
    E = edge_index.shape[1]
    src, dst = edge_index[0], edge_index[1]
    grid = (n // _BN,)
    row = lambda v: v.reshape(1, -1)

    f0 = pre_W0.shape[1]
    A0, B0 = pl.pallas_call(
        _tc_pre_body,
        grid=grid,
        in_specs=[_rows(_BN, fin), _full((fin, f0)), _full((fin, f0)),
                  _full((1, f0))],
        out_specs=[_rows(_BN, f0), _rows(_BN, f0)],
        out_shape=[jax.ShapeDtypeStruct((n, f0), jnp.float32),
                   jax.ShapeDtypeStruct((n, f0), jnp.float32)],
    )(x, pre_W0[:fin], pre_W0[fin:], row(pre_b0))

    (S0,), (Q0,), (M0,), deg = _edge_stats(B0, src, dst, n, f0, E, True, 1)
    deg2 = deg.reshape(n, 1)

    f1 = pre_W1.shape[1]
    pdim0 = post_W0.shape[0]
    h0, A1, B1 = pl.pallas_call(
        _tc_mid_body,
        grid=grid,
        in_specs=[_rows(_BN, fin), _rows(_BN, f0), _rows(_BN, f0),
                  _rows(_BN, f0), _rows(_BN, f0), _rows(_BN, 1),
                  _full((pdim0, f1)), _full((1, f1)), _full((f1, f1)),
                  _full((1, f1)), _full((1, f1)), _full((1, f1)),
                  _full((f1, f1)), _full((f1, f1)), _full((1, f1))],
        out_specs=[_rows(_BN, f1), _rows(_BN, f1), _rows(_BN, f1)],
        out_shape=[jax.ShapeDtypeStruct((n, f1), jnp.float32),
                   jax.ShapeDtypeStruct((n, f1), jnp.float32),
                   jax.ShapeDtypeStruct((n, f1), jnp.float32)],
    )(x, A0, S0, Q0, M0, deg2, post_W0, row(post_b0), lin_W0, row(lin_b0),
      row(ln_g0), row(ln_b0), pre_W1[:f1], pre_W1[f1:], row(pre_b1))

    (S1,), (Q1,), (M1,), _ = _edge_stats(B1, src, dst, n, f1, E, False, 1)

    f2 = lin_W1.shape[1]
    pdim1 = post_W1.shape[0]
    fp = proj_W.shape[1]
    K = hW1.shape[0]
    hh = hW1.shape[2]
    y = pl.pallas_call(
        _tc_out_body,
        grid=grid,
        in_specs=[_rows(_BN, f1), _rows(_BN, f1), _rows(_BN, f1),
                  _rows(_BN, f1), _rows(_BN, f1), _rows(_BN, 1),
                  _rows(_BN, 1),
                  _full((pdim1, f2)), _full((1, f2)), _full((f2, f2)),
                  _full((1, f2)), _full((1, f2)), _full((1, f2)),
                  _full((f2, fp)), _full((1, fp)), _full((1, fp)),
                  _full((1, fp)),
                  _full((K, fp, hh)), _full((K, hh)), _full((K, hh, fp)),
                  _full((K, fp)), _full((K, fp, 1)), _full((K, 1))],
        out_specs=[_rows(_BN, 1)],
        out_shape=[jax.ShapeDtypeStruct((n, 1), jnp.float32)],
    )(h0, A1, S1, Q1, M1, deg2, D.reshape(n, 1), post_W1, row(post_b1),
      lin_W1, row(lin_b1), row(ln_g1), row(ln_b1), proj_W, row(proj_b),
      row(ln_gp), row(ln_bp), hW1, hb1, hW2, hb2, oW, ob)[0]
    return y.reshape(n)
